# baseline (speedup 1.0000x reference)
import functools

import jax
import jax.numpy as jnp
from jax import lax
from jax.experimental import pallas as pl
from jax.experimental.pallas import tpu as pltpu
from jax.experimental.pallas import tpu_sc as plsc

N = 8192
IN_DIM = 64
HID = 128
OUT_DIM = 64
K = 32

RB = 256
NBLK = N // RB
BIG = 3.0e38

NC = 2
NS = 16
NW = NC * NS
ROWS_PER_W = N // NW
SUBROWS = 8
NSUB = ROWS_PER_W // SUBROWS
LANES = 16
HCHUNKS = HID // LANES


NCLS = 128
NE = N // NCLS
CACHE = 6

BIG_I = N


def _extract_topk(s, cols):
    picks = []
    for _ in range(K):
        m = jnp.min(s, axis=1, keepdims=True)
        hit = s == m
        idx = jnp.min(jnp.where(hit, cols, BIG_I), axis=1, keepdims=True)
        picks.append(idx)
        s = jnp.where(hit & (cols == idx), BIG, s)
    return jnp.concatenate(picks, axis=1)


def _xab_body(x_ref, waT_ref, wbT_ref, xa_ref, xb_ref):
    x = x_ref[...]
    xa_ref[...] = jnp.dot(x, waT_ref[...],
                          preferred_element_type=jnp.float32,
                          precision=jax.lax.Precision.HIGHEST)
    xb_ref[...] = jnp.dot(x, wbT_ref[...],
                          preferred_element_type=jnp.float32,
                          precision=jax.lax.Precision.HIGHEST)


def _xab_call(x, waT, wbT):
    return pl.pallas_call(
        _xab_body,
        out_shape=[
            jax.ShapeDtypeStruct((N, HID), jnp.float32),
            jax.ShapeDtypeStruct((N, HID), jnp.float32),
        ],
    )(x, waT, wbT)


def _topk_body(xr_ref, xT_ref, idx_ref, s_ref):
    xr = xr_ref[...]
    xT = xT_ref[...]
    sq = jnp.sum(xT * xT, axis=0, keepdims=True)
    xr_hi = xr.astype(jnp.bfloat16)
    xr_lo = (xr - xr_hi.astype(jnp.float32)).astype(jnp.bfloat16)
    xT_hi = xT.astype(jnp.bfloat16)
    xT_lo = (xT - xT_hi.astype(jnp.float32)).astype(jnp.bfloat16)
    dims = (((1,), (0,)), ((), ()))

    def _mm(a, b):
        return jax.lax.dot_general(a, b, dims,
                                   preferred_element_type=jnp.float32)

    dot = _mm(xr_hi, xT_lo) + _mm(xr_lo, xT_hi) + _mm(xr_hi, xT_hi)
    s_ref[...] = sq - 2.0 * dot

    cvals = [jnp.full((RB, NCLS), BIG, jnp.float32) for _ in range(CACHE)]
    ces = [jnp.zeros((RB, NCLS), jnp.int32) for _ in range(CACHE)]
    for e in range(NE):
        v = s_ref[:, e * NCLS:(e + 1) * NCLS]
        ev = jnp.full((RB, NCLS), e, jnp.int32)
        for t in range(CACHE):
            lt = v < cvals[t]
            cvals[t], v = (jnp.where(lt, v, cvals[t]),
                           jnp.where(lt, cvals[t], v))
            ces[t], ev = (jnp.where(lt, ev, ces[t]),
                          jnp.where(lt, ces[t], ev))

    lane = jax.lax.broadcasted_iota(jnp.int32, (RB, NCLS), 1)
    jcols = [ce * NCLS + lane for ce in ces]
    heads = cvals[0]
    jheads = jcols[0]
    depth = jnp.zeros((RB, NCLS), jnp.int32)
    picks = []
    for _ in range(K):
        m = jnp.min(heads, axis=1, keepdims=True)
        hit = heads == m
        idx = jnp.min(jnp.where(hit, jheads, BIG_I), axis=1, keepdims=True)
        picks.append(idx)
        sel = hit & (jheads == idx)
        pv = jnp.full((RB, NCLS), BIG, jnp.float32)
        pj = jnp.full((RB, NCLS), BIG_I, jnp.int32)
        for d in range(CACHE - 2, -1, -1):
            dmask = depth == d
            pv = jnp.where(dmask, cvals[d + 1], pv)
            pj = jnp.where(dmask, jcols[d + 1], pj)
        heads = jnp.where(sel, pv, heads)
        jheads = jnp.where(sel, pj, jheads)
        depth = jnp.where(sel, depth + 1, depth)
    idx_ref[...] = jnp.concatenate(picks, axis=1)

    flag = jnp.max(jnp.where(depth >= CACHE, 1, 0))

    @pl.when(flag > 0)
    def _fallback():
        cols = jax.lax.broadcasted_iota(jnp.int32, (RB, N), 1)
        idx_ref[...] = _extract_topk(s_ref[...], cols)


def _topk_call(xrows, xT):
    nrows = xrows.shape[0]
    return pl.pallas_call(
        _topk_body,
        grid=(nrows // RB,),
        in_specs=[
            pl.BlockSpec((RB, IN_DIM), lambda i: (i, 0)),
            pl.BlockSpec((IN_DIM, N), lambda i: (0, 0)),
        ],
        out_specs=pl.BlockSpec((RB, K), lambda i: (i, 0)),
        out_shape=jax.ShapeDtypeStruct((nrows, K), jnp.int32),
        scratch_shapes=[pltpu.VMEM((RB, N), jnp.float32)],
    )(xrows, xT)


def _make_pool_body(nrows):
    rows_per_w = nrows // NW
    nsub = rows_per_w // SUBROWS

    def _pool_body(xa_hbm, idx_hbm, xb_hbm, out_hbm,
                   idx_v0, idx_v1, rows_v0, rows_v1, xb_v, out_v, sem0, sem1):
        wid = lax.axis_index("s") * NC + lax.axis_index("c")
        r0 = wid * rows_per_w

        def start_gather(sb, idx_v, rows_v, sem):
            rbase = r0 + sb * SUBROWS
            pltpu.sync_copy(idx_hbm.at[pl.ds(rbase * K, SUBROWS * K)], idx_v)
            pltpu.async_copy(xa_hbm.at[idx_v], rows_v, sem)

        def compute(sb, idx_v, rows_v, sem):
            rbase = r0 + sb * SUBROWS
            pltpu.make_async_copy(xa_hbm.at[idx_v], rows_v, sem).wait()
            pltpu.sync_copy(xb_hbm.at[pl.ds(rbase, SUBROWS)], xb_v)

            def row(r, c2):
                xbs = [xb_v[r, pl.ds(c * LANES, LANES)] for c in range(HCHUNKS)]
                accs = [jnp.zeros((LANES,), jnp.float32) for _ in range(HCHUNKS)]
                for n in range(K):
                    for c in range(HCHUNKS):
                        v = rows_v[r * K + n, pl.ds(c * LANES, LANES)]
                        h = jnp.clip(v + xbs[c], -1.0, 1.0)
                        accs[c] = accs[c] + h
                for c in range(HCHUNKS):
                    out_v[r, pl.ds(c * LANES, LANES)] = accs[c] * (1.0 / K)
                return c2

            lax.fori_loop(0, SUBROWS, row, 0)
            pltpu.sync_copy(out_v, out_hbm.at[pl.ds(rbase, SUBROWS)])

        start_gather(0, idx_v0, rows_v0, sem0)

        def step(i, carry):
            sb0 = 2 * i
            start_gather(sb0 + 1, idx_v1, rows_v1, sem1)
            compute(sb0, idx_v0, rows_v0, sem0)

            @pl.when(sb0 + 2 < nsub)
            def _prefetch():
                start_gather(sb0 + 2, idx_v0, rows_v0, sem0)

            compute(sb0 + 1, idx_v1, rows_v1, sem1)
            return carry

        lax.fori_loop(0, nsub // 2, step, 0)

    return _pool_body


@functools.cache
def _pool_call(nrows):
    return pl.kernel(
        _make_pool_body(nrows),
        out_type=jax.ShapeDtypeStruct((nrows, HID), jnp.float32),
        mesh=plsc.VectorSubcoreMesh(core_axis_name="c", subcore_axis_name="s",
                                    num_cores=NC, num_subcores=NS),
        scratch_types=[
            pltpu.VMEM((SUBROWS * K,), jnp.int32),
            pltpu.VMEM((SUBROWS * K,), jnp.int32),
            pltpu.VMEM((SUBROWS * K, HID), jnp.float32),
            pltpu.VMEM((SUBROWS * K, HID), jnp.float32),
            pltpu.VMEM((SUBROWS, HID), jnp.float32),
            pltpu.VMEM((SUBROWS, HID), jnp.float32),
            pltpu.SemaphoreType.DMA,
            pltpu.SemaphoreType.DMA,
        ],
    )


def _final_body(pool_ref, x_ref, w2pT_ref, w2xT_ref, o_ref):
    o_ref[...] = (
        jnp.dot(pool_ref[...], w2pT_ref[...],
                preferred_element_type=jnp.float32,
                precision=jax.lax.Precision.HIGHEST)
        + jnp.dot(x_ref[...], w2xT_ref[...],
                  preferred_element_type=jnp.float32,
                  precision=jax.lax.Precision.HIGHEST)
    )


def _final_call(pool, x, w2pT, w2xT):
    return pl.pallas_call(
        _final_body,
        out_shape=jax.ShapeDtypeStruct((N, OUT_DIM), jnp.float32),
    )(pool, x, w2pT, w2xT)


def kernel(x, w, w2):
    xT = x.T
    waT = w[:, :IN_DIM].T
    wbT = w[:, IN_DIM:].T
    w2pT = w2[:, :HID].T
    w2xT = w2[:, HID:].T
    xa, xb = _xab_call(x, waT, wbT)
    graph = _topk_call(x, xT)
    pool = _pool_call(N)(xa, graph.reshape(N * K), xb)
    return _final_call(pool, x, w2pT, w2xT)


# speedup vs baseline: 1.0237x; 1.0237x over previous
import functools

import jax
import jax.numpy as jnp
from jax import lax
from jax.experimental import pallas as pl
from jax.experimental.pallas import tpu as pltpu
from jax.experimental.pallas import tpu_sc as plsc

N = 8192
IN_DIM = 64
HID = 128
OUT_DIM = 64
K = 32

RB = 256
NBLK = N // RB
BIG = 3.0e38

NC = 2
NS = 16
NW = NC * NS
ROWS_PER_W = N // NW
SUBROWS = 8
NSUB = ROWS_PER_W // SUBROWS
LANES = 16
HCHUNKS = HID // LANES


NCLS = 128
NE = N // NCLS
CACHE = 6

BIG_I = N


def _extract_topk(s, cols):
    picks = []
    for _ in range(K):
        m = jnp.min(s, axis=1, keepdims=True)
        hit = s == m
        idx = jnp.min(jnp.where(hit, cols, BIG_I), axis=1, keepdims=True)
        picks.append(idx)
        s = jnp.where(hit & (cols == idx), BIG, s)
    return jnp.concatenate(picks, axis=1)


def _xab_body(x_ref, xT_ref, waT_ref, wbT_ref, xa_ref, xb_ref, sq_ref):
    x = x_ref[...]
    xa_ref[...] = jnp.dot(x, waT_ref[...],
                          preferred_element_type=jnp.float32,
                          precision=jax.lax.Precision.HIGHEST)
    xb_ref[...] = jnp.dot(x, wbT_ref[...],
                          preferred_element_type=jnp.float32,
                          precision=jax.lax.Precision.HIGHEST)
    xT = xT_ref[...]
    sq_ref[...] = jnp.sum(xT * xT, axis=0, keepdims=True)


def _xab_call(x, xT, waT, wbT):
    return pl.pallas_call(
        _xab_body,
        out_shape=[
            jax.ShapeDtypeStruct((N, HID), jnp.float32),
            jax.ShapeDtypeStruct((N, HID), jnp.float32),
            jax.ShapeDtypeStruct((1, N), jnp.float32),
        ],
    )(x, xT, waT, wbT)


def _topk_body(xr_ref, xT_ref, sq_ref, idx_ref):
    xr = xr_ref[...]
    xT = xT_ref[...]
    sq = sq_ref[...]
    xr_hi = xr.astype(jnp.bfloat16)
    xr_lo = (xr - xr_hi.astype(jnp.float32)).astype(jnp.bfloat16)
    xT_hi = xT.astype(jnp.bfloat16)
    xT_lo = (xT - xT_hi.astype(jnp.float32)).astype(jnp.bfloat16)
    dims = (((1,), (0,)), ((), ()))

    def _mm(a, b):
        return jax.lax.dot_general(a, b, dims,
                                   preferred_element_type=jnp.float32)

    def _scores():
        dot = _mm(xr_hi, xT_lo) + _mm(xr_lo, xT_hi) + _mm(xr_hi, xT_hi)
        return sq - 2.0 * dot

    s = _scores()

    cvals = [jnp.full((RB, NCLS), BIG, jnp.float32) for _ in range(CACHE)]
    ces = [jnp.zeros((RB, NCLS), jnp.int32) for _ in range(CACHE)]
    for e in range(NE):
        v = s[:, e * NCLS:(e + 1) * NCLS]
        ev = jnp.full((RB, NCLS), e, jnp.int32)
        for t in range(CACHE):
            lt = v < cvals[t]
            cvals[t], v = (jnp.where(lt, v, cvals[t]),
                           jnp.where(lt, cvals[t], v))
            ces[t], ev = (jnp.where(lt, ev, ces[t]),
                          jnp.where(lt, ces[t], ev))

    lane = jax.lax.broadcasted_iota(jnp.int32, (RB, NCLS), 1)
    jcols = [ce * NCLS + lane for ce in ces]
    heads = cvals[0]
    jheads = jcols[0]
    depth = jnp.zeros((RB, NCLS), jnp.int32)
    picks = []
    for _ in range(K):
        m = jnp.min(heads, axis=1, keepdims=True)
        hit = heads == m
        idx = jnp.min(jnp.where(hit, jheads, BIG_I), axis=1, keepdims=True)
        picks.append(idx)
        sel = hit & (jheads == idx)
        pv = jnp.full((RB, NCLS), BIG, jnp.float32)
        pj = jnp.full((RB, NCLS), BIG_I, jnp.int32)
        for d in range(CACHE - 2, -1, -1):
            dmask = depth == d
            pv = jnp.where(dmask, cvals[d + 1], pv)
            pj = jnp.where(dmask, jcols[d + 1], pj)
        heads = jnp.where(sel, pv, heads)
        jheads = jnp.where(sel, pj, jheads)
        depth = jnp.where(sel, depth + 1, depth)
    idx_ref[...] = jnp.concatenate(picks, axis=1)

    flag = jnp.max(jnp.where(depth >= CACHE, 1, 0))

    @pl.when(flag > 0)
    def _fallback():
        cols = jax.lax.broadcasted_iota(jnp.int32, (RB, N), 1)
        idx_ref[...] = _extract_topk(_scores(), cols)


def _topk_call(xrows, xT, sq):
    nrows = xrows.shape[0]
    return pl.pallas_call(
        _topk_body,
        grid=(nrows // RB,),
        in_specs=[
            pl.BlockSpec((RB, IN_DIM), lambda i: (i, 0)),
            pl.BlockSpec((IN_DIM, N), lambda i: (0, 0)),
            pl.BlockSpec((1, N), lambda i: (0, 0)),
        ],
        out_specs=pl.BlockSpec((RB, K), lambda i: (i, 0)),
        out_shape=jax.ShapeDtypeStruct((nrows, K), jnp.int32),
    )(xrows, xT, sq)


def _make_pool_body(nrows):
    rows_per_w = nrows // NW
    nsub = rows_per_w // SUBROWS

    def _pool_body(xa_hbm, idx_hbm, xb_hbm, out_hbm,
                   idx_v0, idx_v1, rows_v0, rows_v1, xb_v, out_v, sem0, sem1):
        wid = lax.axis_index("s") * NC + lax.axis_index("c")
        r0 = wid * rows_per_w

        def start_gather(sb, idx_v, rows_v, sem):
            rbase = r0 + sb * SUBROWS
            pltpu.sync_copy(idx_hbm.at[pl.ds(rbase * K, SUBROWS * K)], idx_v)
            pltpu.async_copy(xa_hbm.at[idx_v], rows_v, sem)

        def compute(sb, idx_v, rows_v, sem):
            rbase = r0 + sb * SUBROWS
            pltpu.make_async_copy(xa_hbm.at[idx_v], rows_v, sem).wait()
            pltpu.sync_copy(xb_hbm.at[pl.ds(rbase, SUBROWS)], xb_v)

            def row(r, c2):
                xbs = [xb_v[r, pl.ds(c * LANES, LANES)] for c in range(HCHUNKS)]
                accs = [jnp.zeros((LANES,), jnp.float32) for _ in range(HCHUNKS)]
                for n in range(K):
                    for c in range(HCHUNKS):
                        v = rows_v[r * K + n, pl.ds(c * LANES, LANES)]
                        h = jnp.clip(v + xbs[c], -1.0, 1.0)
                        accs[c] = accs[c] + h
                for c in range(HCHUNKS):
                    out_v[r, pl.ds(c * LANES, LANES)] = accs[c] * (1.0 / K)
                return c2

            lax.fori_loop(0, SUBROWS, row, 0)
            pltpu.sync_copy(out_v, out_hbm.at[pl.ds(rbase, SUBROWS)])

        start_gather(0, idx_v0, rows_v0, sem0)

        def step(i, carry):
            sb0 = 2 * i
            start_gather(sb0 + 1, idx_v1, rows_v1, sem1)
            compute(sb0, idx_v0, rows_v0, sem0)

            @pl.when(sb0 + 2 < nsub)
            def _prefetch():
                start_gather(sb0 + 2, idx_v0, rows_v0, sem0)

            compute(sb0 + 1, idx_v1, rows_v1, sem1)
            return carry

        lax.fori_loop(0, nsub // 2, step, 0)

    return _pool_body


@functools.cache
def _pool_call(nrows):
    return pl.kernel(
        _make_pool_body(nrows),
        out_type=jax.ShapeDtypeStruct((nrows, HID), jnp.float32),
        mesh=plsc.VectorSubcoreMesh(core_axis_name="c", subcore_axis_name="s",
                                    num_cores=NC, num_subcores=NS),
        scratch_types=[
            pltpu.VMEM((SUBROWS * K,), jnp.int32),
            pltpu.VMEM((SUBROWS * K,), jnp.int32),
            pltpu.VMEM((SUBROWS * K, HID), jnp.float32),
            pltpu.VMEM((SUBROWS * K, HID), jnp.float32),
            pltpu.VMEM((SUBROWS, HID), jnp.float32),
            pltpu.VMEM((SUBROWS, HID), jnp.float32),
            pltpu.SemaphoreType.DMA,
            pltpu.SemaphoreType.DMA,
        ],
    )


def _final_body(pool_ref, x_ref, w2pT_ref, w2xT_ref, o_ref):
    o_ref[...] = (
        jnp.dot(pool_ref[...], w2pT_ref[...],
                preferred_element_type=jnp.float32,
                precision=jax.lax.Precision.HIGHEST)
        + jnp.dot(x_ref[...], w2xT_ref[...],
                  preferred_element_type=jnp.float32,
                  precision=jax.lax.Precision.HIGHEST)
    )


def _final_call(pool, x, w2pT, w2xT):
    return pl.pallas_call(
        _final_body,
        out_shape=jax.ShapeDtypeStruct((N, OUT_DIM), jnp.float32),
    )(pool, x, w2pT, w2xT)


def kernel(x, w, w2):
    xT = x.T
    waT = w[:, :IN_DIM].T
    wbT = w[:, IN_DIM:].T
    w2pT = w2[:, :HID].T
    w2xT = w2[:, HID:].T
    xa, xb, sq = _xab_call(x, xT, waT, wbT)
    half = N // 2
    g1 = _topk_call(x[:half], xT, sq)
    p1 = _pool_call(half)(xa, g1.reshape(half * K), xb[:half])
    g2 = _topk_call(x[half:], xT, sq)
    p2 = _pool_call(half)(xa, g2.reshape(half * K), xb[half:])
    pool = jnp.concatenate([p1, p2], axis=0)
    return _final_call(pool, x, w2pT, w2xT)
